# baseline (speedup 1.0000x reference)
import functools
import math

import jax
import jax.numpy as jnp
from jax import lax
from jax.experimental import pallas as pl
from jax.experimental.pallas import tpu as pltpu
from jax.experimental.pallas import tpu_sc as plsc

V = 65
VP = 128
RP = 72
MP = 40
B = 64
T = 512
C = 128
NCH = T // C
L = 16
NW = 32
LOG_W = math.log(1.0 / 3.0 + 1e-10)
LN2 = 0.6931471805599453
SENT_TB1 = 65 * RP
SENT_TB0 = 65 * RP + 65
SENT_FOUR = 33


def _softlog(x):
    bits = lax.bitcast_convert_type(x, jnp.int32)
    e = (bits >> 23) - 127
    m = lax.bitcast_convert_type((bits & 0x7FFFFF) | (127 << 23), jnp.float32)
    s = (m - 1.0) / (m + 1.0)
    z = s * s
    p = s * (2.0 + z * (0.6666666666666666 + z * 0.4))
    return e.astype(jnp.float32) * LN2 + p


def _sc_body(idx_ref, tb_ref, four_ref, out_ref,
             idx_v, rtb_v, rfour_v, par_v,
             tb_rows, four_rows, out_v, sem_g, sem_o):
    cid = lax.axis_index("c")
    sid = lax.axis_index("s")
    wid = sid * 2 + cid
    rows_per_tile = B // NW

    def idx_calc(ci):
        p = ci % 2
        tok0 = ci * C
        for g in range(C // L):
            tvec = lax.iota(jnp.int32, L) + (tok0 + g * L)
            i1 = plsc.load_gather(idx_v, [jnp.maximum(tvec - 1, 0)])
            i2 = plsc.load_gather(idx_v, [jnp.maximum(tvec - 2, 0)])
            i3 = plsc.load_gather(idx_v, [jnp.maximum(tvec - 3, 0)])
            rtb = i2 * RP + i1
            rfour = (i3 * V + i2) * MP + (i1 >> 1)
            par = i1 & 1
            if ci == 0 and g == 0:
                rtb = jnp.where(tvec >= 2, rtb,
                                jnp.where(tvec == 1, SENT_TB1 + i1, SENT_TB0))
                rfour = jnp.where(tvec >= 3, rfour, SENT_FOUR)
                par = jnp.where(tvec >= 3, par, 0)
            rtb_v[p, pl.ds(g * L, L)] = rtb
            rfour_v[p, pl.ds(g * L, L)] = rfour
            par_v[p, pl.ds(g * L, L)] = par

    def issue_gathers(ci):
        p = ci % 2
        return (
            pltpu.async_copy(tb_ref.at[rtb_v.at[p]], tb_rows.at[p], sem_g.at[p]),
            pltpu.async_copy(four_ref.at[rfour_v.at[p]], four_rows.at[p], sem_g.at[p]),
        )

    for rr in range(rows_per_tile):
        b = wid * rows_per_tile + rr
        pltpu.sync_copy(idx_ref.at[b], idx_v)

        idx_calc(0)
        cps = {0: issue_gathers(0)}
        out_cps = {}
        for ci in range(NCH):
            p = ci % 2
            if ci + 1 < NCH:
                idx_calc(ci + 1)
                cps[ci + 1] = issue_gathers(ci + 1)
            for cp in cps.pop(ci):
                cp.wait()
            if ci >= 2:
                out_cps.pop(ci - 2).wait()

            def tok_body(t, _, p=p):
                lanes = lax.iota(jnp.int32, L)
                odd = plsc.load_gather(par_v.at[p], [lanes * 0 + t]) > 0
                for j in range(5):
                    sl = pl.ds(j * L, L)
                    w = four_rows[p, t, sl]
                    f4 = lax.bitcast_convert_type(
                        jnp.where(odd, w & jnp.int32(-65536), w << 16),
                        jnp.float32)
                    s = tb_rows[p, t, sl] + jnp.exp(f4)
                    out_v[p, t, sl] = _softlog(s) + LOG_W
                return 0

            lax.fori_loop(0, C, tok_body, 0)
            out_cps[ci] = pltpu.async_copy(
                out_v.at[p], out_ref.at[pl.ds(b * T + ci * C, C)], sem_o.at[p])
        for ci in sorted(out_cps):
            out_cps.pop(ci).wait()


@functools.partial(
    pl.kernel,
    mesh=plsc.VectorSubcoreMesh(core_axis_name="c", subcore_axis_name="s"),
    out_type=jax.ShapeDtypeStruct((B * T, VP), jnp.float32),
    compiler_params=pltpu.CompilerParams(needs_layout_passes=False),
    scratch_types=[
        pltpu.VMEM((T,), jnp.int32),
        pltpu.VMEM((2, C), jnp.int32),
        pltpu.VMEM((2, C), jnp.int32),
        pltpu.VMEM((2, C), jnp.int32),
        pltpu.VMEM((2, C, VP), jnp.float32),
        pltpu.VMEM((2, C, VP), jnp.int32),
        pltpu.VMEM((2, C, VP), jnp.float32),
        pltpu.SemaphoreType.DMA((2,)),
        pltpu.SemaphoreType.DMA((2,)),
    ],
)
def _ngram_sc_kernel(*refs):
    _sc_body(*refs)


def _pack_body(four_ref, out_ref):
    for b in range(V):
        bits = lax.bitcast_convert_type(four_ref[0, b], jnp.int32)
        bb = (bits + 0x7FFF + ((bits >> 16) & 1)) >> 16
        bb = jnp.concatenate([bb, jnp.zeros((1, V), jnp.int32)], axis=0)
        b3 = bb.reshape(MP - 7, 2, V)
        w = (b3[:, 0, :] & 0xFFFF) | (b3[:, 1, :] << 16)
        out_ref[pl.ds(b * MP, MP - 7), pl.ds(0, V)] = w

    @pl.when(pl.program_id(0) == 0)
    def _zero_sentinel():
        out_ref[pl.ds(MP - 7, 7), :] = jnp.zeros((7, VP), jnp.int32)


_pack_fourgram = pl.pallas_call(
    _pack_body,
    grid=(V,),
    in_specs=[pl.BlockSpec((1, V, V, V), lambda a: (a, 0, 0, 0))],
    out_specs=pl.BlockSpec((V * MP, VP), lambda a: (a, 0)),
    out_shape=jax.ShapeDtypeStruct((V * V * MP, VP), jnp.int32),
)


def kernel(idx, bigram_log_probs, trigram_log_probs, fourgram_log_probs):
    idx32 = idx.astype(jnp.int32)
    eb = jnp.exp(bigram_log_probs)
    tb = jnp.exp(trigram_log_probs) + eb[None, :, :]
    tb = jnp.concatenate([tb, (eb + 1.0)[None, :, :]], axis=0)
    tb_p = jnp.pad(tb, ((0, 0), (0, RP - V), (0, VP - V)))
    tb_p = tb_p.at[V, V, :].set(2.0)
    tb_p = tb_p.reshape((V + 1) * RP, VP)
    four_p = _pack_fourgram(fourgram_log_probs)
    out = _ngram_sc_kernel(idx32, tb_p, four_p)
    return out[:, :V].reshape(B, T, V)


# speedup vs baseline: 1.2735x; 1.2735x over previous
import functools
import math

import jax
import jax.numpy as jnp
from jax import lax
from jax.experimental import pallas as pl
from jax.experimental.pallas import tpu as pltpu
from jax.experimental.pallas import tpu_sc as plsc

V = 65
VP = 128
RP = 72
MP = 33
AP = 2152
B = 64
T = 512
C = 128
NCH = T // C
L = 16
NW = 32
LOG_W = math.log(1.0 / 3.0 + 1e-10)
LN2 = 0.6931471805599453
SENT_TB1 = 65 * RP
SENT_TB0 = 65 * RP + 65
SENT_FOUR = 65 * MP


def _softlog(x):
    bits = lax.bitcast_convert_type(x, jnp.int32)
    e = (bits >> 23) - 127
    m = lax.bitcast_convert_type((bits & 0x7FFFFF) | (127 << 23), jnp.float32)
    s = (m - 1.0) / (m + 1.0)
    z = s * s
    p = s * (2.0 + z * (0.6666666666666666 + z * 0.4))
    return e.astype(jnp.float32) * LN2 + p


def _sc_body(idx_ref, tb_ref, four_ref, out_ref,
             idx_v, rtb_v, rfour_v, par_v,
             tb_rows, four_rows, out_v, sem_g, sem_o):
    cid = lax.axis_index("c")
    sid = lax.axis_index("s")
    wid = sid * 2 + cid
    rows_per_tile = B // NW

    def idx_calc(ci):
        p = ci % 2
        tok0 = ci * C
        for g in range(C // L):
            tvec = lax.iota(jnp.int32, L) + (tok0 + g * L)
            i1 = plsc.load_gather(idx_v, [jnp.maximum(tvec - 1, 0)])
            i2 = plsc.load_gather(idx_v, [jnp.maximum(tvec - 2, 0)])
            i3 = plsc.load_gather(idx_v, [jnp.maximum(tvec - 3, 0)])
            rtb = i2 * RP + i1
            u = jnp.where(i1 >= MP, i1 - MP, i1)
            rfour = i3 * AP + i2 * MP + u
            par = (i1 >= MP).astype(jnp.int32)
            if ci == 0 and g == 0:
                rtb = jnp.where(tvec >= 2, rtb,
                                jnp.where(tvec == 1, SENT_TB1 + i1, SENT_TB0))
                rfour = jnp.where(tvec >= 3, rfour, SENT_FOUR)
                par = jnp.where(tvec >= 3, par, 0)
            rtb_v[p, pl.ds(g * L, L)] = rtb
            rfour_v[p, pl.ds(g * L, L)] = rfour
            par_v[p, pl.ds(g * L, L)] = par

    def issue_gathers(ci):
        p = ci % 2
        return (
            pltpu.async_copy(tb_ref.at[rtb_v.at[p]], tb_rows.at[p], sem_g.at[p]),
            pltpu.async_copy(four_ref.at[rfour_v.at[p]], four_rows.at[p], sem_g.at[p]),
        )

    for rr in range(rows_per_tile):
        b = wid * rows_per_tile + rr
        pltpu.sync_copy(idx_ref.at[b], idx_v)

        idx_calc(0)
        cps = {0: issue_gathers(0)}
        out_cps = {}
        for ci in range(NCH):
            p = ci % 2
            if ci + 1 < NCH:
                idx_calc(ci + 1)
                cps[ci + 1] = issue_gathers(ci + 1)
            for cp in cps.pop(ci):
                cp.wait()
            if ci >= 2:
                out_cps.pop(ci - 2).wait()

            def tok_body(t, _, p=p):
                lanes = lax.iota(jnp.int32, L)
                odd = plsc.load_gather(par_v.at[p], [lanes * 0 + t]) > 0
                for j in range(5):
                    sl = pl.ds(j * L, L)
                    w = four_rows[p, t, sl]
                    f4 = lax.bitcast_convert_type(
                        jnp.where(odd, w & jnp.int32(-65536), w << 16),
                        jnp.float32)
                    s = tb_rows[p, t, sl] + jnp.exp(f4)
                    out_v[p, t, sl] = _softlog(s) + LOG_W
                return 0

            lax.fori_loop(0, C, tok_body, 0)
            out_cps[ci] = pltpu.async_copy(
                out_v.at[p], out_ref.at[pl.ds(b * T + ci * C, C)], sem_o.at[p])
        for ci in sorted(out_cps):
            out_cps.pop(ci).wait()


@functools.partial(
    pl.kernel,
    mesh=plsc.VectorSubcoreMesh(core_axis_name="c", subcore_axis_name="s"),
    out_type=jax.ShapeDtypeStruct((B * T, VP), jnp.float32),
    compiler_params=pltpu.CompilerParams(needs_layout_passes=False),
    scratch_types=[
        pltpu.VMEM((T,), jnp.int32),
        pltpu.VMEM((2, C), jnp.int32),
        pltpu.VMEM((2, C), jnp.int32),
        pltpu.VMEM((2, C), jnp.int32),
        pltpu.VMEM((2, C, VP), jnp.float32),
        pltpu.VMEM((2, C, VP), jnp.int32),
        pltpu.VMEM((2, C, VP), jnp.float32),
        pltpu.SemaphoreType.DMA((2,)),
        pltpu.SemaphoreType.DMA((2,)),
    ],
)
def _ngram_sc_kernel(*refs):
    _sc_body(*refs)


def _pack_body(four_ref, out_ref):
    bits = lax.bitcast_convert_type(four_ref[0], jnp.int32)
    bb = (bits + 0x7FFF + ((bits >> 16) & 1)) >> 16
    lo = bb[:, 0:MP, :] & 0xFFFF
    hi = bb[:, MP:V, :] << 16
    w = lo | jnp.pad(hi, ((0, 0), (0, 1), (0, 0)))
    for b in range(V):
        out_ref[pl.ds(b * MP, MP), pl.ds(0, V)] = w[b]

    @pl.when(pl.program_id(0) == 0)
    def _zero_sentinel():
        out_ref[pl.ds(V * MP, AP - V * MP), :] = jnp.zeros(
            (AP - V * MP, VP), jnp.int32)


_pack_fourgram = pl.pallas_call(
    _pack_body,
    grid=(V,),
    in_specs=[pl.BlockSpec((1, V, V, V), lambda a: (a, 0, 0, 0))],
    out_specs=pl.BlockSpec((AP, VP), lambda a: (a, 0)),
    out_shape=jax.ShapeDtypeStruct((V * AP, VP), jnp.int32),
)


def kernel(idx, bigram_log_probs, trigram_log_probs, fourgram_log_probs):
    idx32 = idx.astype(jnp.int32)
    eb = jnp.exp(bigram_log_probs)
    tb = jnp.exp(trigram_log_probs) + eb[None, :, :]
    tb = jnp.concatenate([tb, (eb + 1.0)[None, :, :]], axis=0)
    tb_p = jnp.pad(tb, ((0, 0), (0, RP - V), (0, VP - V)))
    tb_p = tb_p.at[V, V, :].set(2.0)
    tb_p = tb_p.reshape((V + 1) * RP, VP)
    four_p = _pack_fourgram(fourgram_log_probs)
    out = _ngram_sc_kernel(idx32, tb_p, four_p)
    return out[:, :V].reshape(B, T, V)


# speedup vs baseline: 1.7647x; 1.3857x over previous
import functools
import math

import jax
import jax.numpy as jnp
from jax import lax
from jax.experimental import pallas as pl
from jax.experimental.pallas import tpu as pltpu
from jax.experimental.pallas import tpu_sc as plsc

V = 65
VP = 128
RP = 72
B = 64
T = 512
C = 128
NCH = T // C
L = 16
NW = 32
LOG_W = math.log(1.0 / 3.0 + 1e-10)
LN2 = 0.6931471805599453
SQRT2 = 1.4142135623730951
SENT_TB1 = 65 * RP
SENT_TB0 = 65 * RP + 65
SENT_FOUR = 65


def _softlog(x):
    bits = lax.bitcast_convert_type(x, jnp.int32)
    e = (bits >> 23) - 127
    m = lax.bitcast_convert_type((bits & 0x7FFFFF) | (127 << 23), jnp.float32)
    s = (m - 1.0) / (m + 1.0)
    z = s * s
    p = s * (2.0 + z * (0.6666666666666666 + z * 0.4))
    return e.astype(jnp.float32) * LN2 + p


def _sc_body(idx_ref, tb_ref, four_ref, out_ref,
             idx_v, rtb_v, rfour_v,
             tb_rows, four_rows, out_v, sem_g, sem_o):
    cid = lax.axis_index("c")
    sid = lax.axis_index("s")
    wid = sid * 2 + cid
    rows_per_tile = B // NW

    def idx_calc(ci):
        p = ci % 2
        tok0 = ci * C
        for g in range(C // L):
            tvec = lax.iota(jnp.int32, L) + (tok0 + g * L)
            i1 = plsc.load_gather(idx_v, [jnp.maximum(tvec - 1, 0)])
            i2 = plsc.load_gather(idx_v, [jnp.maximum(tvec - 2, 0)])
            i3 = plsc.load_gather(idx_v, [jnp.maximum(tvec - 3, 0)])
            rtb = i2 * RP + i1
            rfour = (i3 * V + i2) * RP + i1
            if ci == 0 and g == 0:
                rtb = jnp.where(tvec >= 2, rtb,
                                jnp.where(tvec == 1, SENT_TB1 + i1, SENT_TB0))
                rfour = jnp.where(tvec >= 3, rfour, SENT_FOUR)
            rtb_v[p, pl.ds(g * L, L)] = rtb
            rfour_v[p, pl.ds(g * L, L)] = rfour

    def issue_gathers(ci):
        p = ci % 2
        return (
            pltpu.async_copy(tb_ref.at[rtb_v.at[p]], tb_rows.at[p], sem_g.at[p]),
            pltpu.async_copy(four_ref.at[rfour_v.at[p]], four_rows.at[p], sem_g.at[p]),
        )

    for rr in range(rows_per_tile):
        b = wid * rows_per_tile + rr
        pltpu.sync_copy(idx_ref.at[b], idx_v)

        idx_calc(0)
        cps = {0: issue_gathers(0)}
        out_cps = {}
        for ci in range(NCH):
            p = ci % 2
            if ci + 1 < NCH:
                idx_calc(ci + 1)
                cps[ci + 1] = issue_gathers(ci + 1)
            for cp in cps.pop(ci):
                cp.wait()
            if ci >= 2:
                out_cps.pop(ci - 2).wait()

            def tok_body(t, _, p=p):
                for j in range(5):
                    sl = pl.ds(j * L, L)
                    s = tb_rows[p, t, sl] + jnp.exp(four_rows[p, t, sl])
                    out_v[p, t, sl] = _softlog(s) + LOG_W
                return 0

            lax.fori_loop(0, C, tok_body, 0)
            out_cps[ci] = pltpu.async_copy(
                out_v.at[p], out_ref.at[pl.ds(b * T + ci * C, C)], sem_o.at[p])
        for ci in sorted(out_cps):
            out_cps.pop(ci).wait()


@functools.partial(
    pl.kernel,
    mesh=plsc.VectorSubcoreMesh(core_axis_name="c", subcore_axis_name="s"),
    out_type=jax.ShapeDtypeStruct((B * T, VP), jnp.float32),
    compiler_params=pltpu.CompilerParams(needs_layout_passes=False),
    scratch_types=[
        pltpu.VMEM((T,), jnp.int32),
        pltpu.VMEM((2, C), jnp.int32),
        pltpu.VMEM((2, C), jnp.int32),
        pltpu.VMEM((2, C, VP), jnp.float32),
        pltpu.VMEM((2, C, VP), jnp.float32),
        pltpu.VMEM((2, C, VP), jnp.float32),
        pltpu.SemaphoreType.DMA((2,)),
        pltpu.SemaphoreType.DMA((2,)),
    ],
)
def _ngram_sc_kernel(*refs):
    _sc_body(*refs)


def _pad_body(four_ref, out_ref):
    for b in range(V):
        out_ref[pl.ds(b * RP, V), pl.ds(0, V)] = four_ref[0, b]
    @pl.when(pl.program_id(0) == 0)
    def _zero_sentinel():
        out_ref[pl.ds(V, RP - V), :] = jnp.zeros((RP - V, VP), jnp.float32)


_pad_fourgram = pl.pallas_call(
    _pad_body,
    grid=(V,),
    in_specs=[pl.BlockSpec((1, V, V, V), lambda a: (a, 0, 0, 0))],
    out_specs=pl.BlockSpec((V * RP, VP), lambda a: (a, 0)),
    out_shape=jax.ShapeDtypeStruct((V * V * RP, VP), jnp.float32),
)


def kernel(idx, bigram_log_probs, trigram_log_probs, fourgram_log_probs):
    idx32 = idx.astype(jnp.int32)
    eb = jnp.exp(bigram_log_probs)
    tb = jnp.exp(trigram_log_probs) + eb[None, :, :]
    tb = jnp.concatenate([tb, (eb + 1.0)[None, :, :]], axis=0)
    tb_p = jnp.pad(tb, ((0, 0), (0, RP - V), (0, VP - V)))
    tb_p = tb_p.at[V, V, :].set(2.0)
    tb_p = tb_p.reshape((V + 1) * RP, VP)
    four_p = _pad_fourgram(fourgram_log_probs)
    out = _ngram_sc_kernel(idx32, tb_p, four_p)
    return out[:, :V].reshape(B, T, V)
